# Initial kernel scaffold; baseline (speedup 1.0000x reference)
import jax
import jax.numpy as jnp
from jax.experimental import pallas as pl


def kernel(delta_t, W, b):
    raise NotImplementedError("write your pallas kernel here")


# speedup vs baseline: 1.5753x; 1.5753x over previous
import functools
import math

import jax
import jax.numpy as jnp
from jax import lax
from jax.experimental import pallas as pl
from jax.experimental.pallas import tpu as pltpu
from jax.experimental.pallas import tpu_sc as plsc

DIM = 256
T_MAX = 240

NC = 2
NS = 16
NW = NC * NS

B_TOTAL = 4096 * 50
ROWS_PER_W = B_TOTAL // NW
CHUNK = 128
NCHUNK = ROWS_PER_W // CHUNK


def _build_base():
    t = jnp.arange(T_MAX, dtype=jnp.float32)[:, None]
    denominator = jnp.exp(
        jnp.arange(DIM, dtype=jnp.float32) * math.log(10000.0) / DIM)
    base = t / denominator
    col = jnp.arange(DIM)
    return jnp.where((col % 2) == 0, jnp.sin(base), jnp.cos(base))


def _proj_body(base_ref, wt_ref, b_ref, out_ref):
    out_ref[...] = jnp.dot(
        base_ref[...], wt_ref[...],
        preferred_element_type=jnp.float32) + b_ref[...]


def _project_table(base, Wt, b2):
    return pl.pallas_call(
        _proj_body,
        out_shape=jax.ShapeDtypeStruct((T_MAX, DIM), jnp.float32),
    )(base, Wt, b2)


_MESH = plsc.VectorSubcoreMesh(
    core_axis_name="c", subcore_axis_name="s", num_cores=NC, num_subcores=NS)


@functools.partial(
    pl.kernel,
    out_type=jax.ShapeDtypeStruct((B_TOTAL, DIM), jnp.float32),
    mesh=_MESH,
    scratch_types=[
        pltpu.VMEM((NCHUNK, CHUNK), jnp.int32),
        pltpu.VMEM((CHUNK, DIM), jnp.float32),
        pltpu.SemaphoreType.DMA,
    ],
)
def _gather(table_hbm, idx_hbm, out_hbm, idx_v, buf, gsem):
    wid = lax.axis_index("s") * NC + lax.axis_index("c")
    pltpu.sync_copy(idx_hbm.at[wid], idx_v)
    row0 = wid * ROWS_PER_W

    def step(j, _):
        pltpu.async_copy(table_hbm.at[idx_v.at[j]], buf, gsem).wait()
        pltpu.sync_copy(buf, out_hbm.at[pl.ds(row0 + j * CHUNK, CHUNK)])
        return 0

    lax.fori_loop(0, NCHUNK, step, 0)


def kernel(delta_t, W, b):
    B, L = delta_t.shape
    base = _build_base()
    proj = _project_table(base, W.T, b[None, :])
    idx = delta_t.reshape(-1).astype(jnp.int32).reshape(NW, NCHUNK, CHUNK)
    out = _gather(proj, idx)
    return out.reshape(B, L, DIM)
